# baseline (speedup 1.0000x reference)
import functools

import jax
import jax.numpy as jnp
from jax import lax
from jax.experimental import pallas as pl
from jax.experimental.pallas import tpu as pltpu
from jax.experimental.pallas import tpu_sc as plsc

VOCAB = 100000
D_MODEL = 64

_NC = 2
_NS = 16
_NW = _NC * _NS
_L = 16

_NSEQ = 4096
_SEQLEN = 200
_S_PER_W = _NSEQ // _NW
_NSB = _S_PER_W // _L


def _emb_body(seq_hbm, table_hbm, out_hbm,
              idx_v, idxt_v, rows_v, tbuf_v, gsem, osem):
    wid = lax.axis_index("s") * _NC + lax.axis_index("c")
    s0 = wid * _S_PER_W

    pltpu.sync_copy(seq_hbm.at[pl.ds(s0, _S_PER_W)], idx_v)

    lanes = lax.iota(jnp.int32, _L)
    rvecs = [lanes + (sb * _L) for sb in range(_NSB)]

    def start_gather(c, b):
        col = jnp.full((_L,), c, jnp.int32)
        for sb in range(_NSB):
            idxt_v[b, pl.ds(sb * _L, _L)] = plsc.load_gather(
                idx_v, [rvecs[sb], col])
        pltpu.async_copy(table_hbm.at[idxt_v.at[b]], rows_v.at[b],
                         gsem.at[b])

    def wait_gather(b):
        pltpu.make_async_copy(table_hbm.at[idxt_v.at[b]], rows_v.at[b],
                              gsem.at[b]).wait()

    def transpose(b):
        for d in range(D_MODEL):
            dcol = jnp.full((_L,), d, jnp.int32)
            for sb in range(_NSB):
                tbuf_v[b, d, pl.ds(sb * _L, _L)] = plsc.load_gather(
                    rows_v.at[b], [rvecs[sb], dcol])

    def out_slab(c):
        return out_hbm.at[c, :, pl.ds(s0, _S_PER_W)]

    def start_store(c, b):
        pltpu.async_copy(tbuf_v.at[b], out_slab(c), osem.at[b])

    def wait_store(c, b):
        pltpu.make_async_copy(tbuf_v.at[b], out_slab(c), osem.at[b]).wait()

    start_gather(0, 0)
    start_gather(1, 1)

    def outer(o, carry):
        for b in range(2):
            c = 2 * o + b
            wait_gather(b)

            @pl.when(o >= 1)
            def _():
                wait_store(c, b)

            transpose(b)
            start_store(c, b)

            @pl.when(c + 2 < _SEQLEN)
            def _():
                start_gather(c + 2, b)

        return carry

    lax.fori_loop(0, _SEQLEN // 2, outer, 0)

    for b in range(2):
        wait_store(_SEQLEN - 2 + b, b)


_emb = functools.partial(
    pl.kernel,
    out_type=jax.ShapeDtypeStruct((_SEQLEN, D_MODEL, _NSEQ), jnp.float32),
    mesh=plsc.VectorSubcoreMesh(core_axis_name="c", subcore_axis_name="s"),
    scratch_types=[
        pltpu.VMEM((_S_PER_W, _SEQLEN), jnp.int32),
        pltpu.VMEM((2, _S_PER_W), jnp.int32),
        pltpu.VMEM((2, _S_PER_W, D_MODEL), jnp.float32),
        pltpu.VMEM((2, D_MODEL, _S_PER_W), jnp.float32),
        pltpu.SemaphoreType.DMA((2,)),
        pltpu.SemaphoreType.DMA((2,)),
    ],
    compiler_params=pltpu.CompilerParams(use_tc_tiling_on_sc=False,
                                         needs_layout_passes=False),
)(_emb_body)


def kernel(sequence, lookup):
    out_t = _emb(sequence.astype(jnp.int32), lookup)
    return jnp.transpose(out_t, (2, 0, 1))


# speedup vs baseline: 2.1760x; 2.1760x over previous
import functools

import jax
import jax.numpy as jnp
from jax import lax
from jax.experimental import pallas as pl
from jax.experimental.pallas import tpu as pltpu
from jax.experimental.pallas import tpu_sc as plsc

VOCAB = 100000
D_MODEL = 64

_NC = 2
_NS = 16
_NW = _NC * _NS

_B = 4096 * 200
_B_PER_W = _B // _NW
_CHUNK = 256
_N_CHUNK = _B_PER_W // _CHUNK
_NBUF = 4
_N_OUTER = _N_CHUNK // _NBUF


def _emb_body(idx_hbm, table_hbm, out_hbm, idx_v, rows_v, gsem, osem):
    wid = lax.axis_index("s") * _NC + lax.axis_index("c")
    base = wid * _B_PER_W

    def start_gather(c, b):
        off = base + c * _CHUNK
        pltpu.sync_copy(idx_hbm.at[pl.ds(off, _CHUNK)], idx_v.at[b])
        pltpu.async_copy(table_hbm.at[idx_v.at[b]], rows_v.at[b], gsem.at[b])

    def wait_gather(b):
        pltpu.make_async_copy(
            table_hbm.at[idx_v.at[b]], rows_v.at[b], gsem.at[b]).wait()

    def start_store(c, b):
        off = base + c * _CHUNK
        pltpu.async_copy(rows_v.at[b], out_hbm.at[pl.ds(off, _CHUNK)],
                         osem.at[b])

    def wait_store(c, b):
        off = base + c * _CHUNK
        pltpu.make_async_copy(
            rows_v.at[b], out_hbm.at[pl.ds(off, _CHUNK)], osem.at[b]).wait()

    for b in range(_NBUF):
        start_gather(b, b)

    def outer(o, carry):
        c0 = o * _NBUF
        for b in range(_NBUF):
            wait_gather(b)
            start_store(c0 + b, b)
        for b in range(_NBUF):
            wait_store(c0 + b, b)
            start_gather(c0 + b + _NBUF, b)
        return carry

    lax.fori_loop(0, _N_OUTER - 1, outer, 0)

    c0 = (_N_OUTER - 1) * _NBUF
    for b in range(_NBUF):
        wait_gather(b)
        start_store(c0 + b, b)
    for b in range(_NBUF):
        wait_store(c0 + b, b)


_emb = functools.partial(
    pl.kernel,
    out_type=jax.ShapeDtypeStruct((_B, D_MODEL), jnp.float32),
    mesh=plsc.VectorSubcoreMesh(core_axis_name="c", subcore_axis_name="s"),
    scratch_types=[
        pltpu.VMEM((_NBUF, _CHUNK), jnp.int32),
        pltpu.VMEM((_NBUF, _CHUNK, D_MODEL), jnp.float32),
        pltpu.SemaphoreType.DMA((_NBUF,)),
        pltpu.SemaphoreType.DMA((_NBUF,)),
    ],
    compiler_params=pltpu.CompilerParams(use_tc_tiling_on_sc=False),
)(_emb_body)


def kernel(sequence, lookup):
    idx = sequence.reshape(-1).astype(jnp.int32)
    out = _emb(idx, lookup)
    return out.reshape(sequence.shape + (D_MODEL,))


# speedup vs baseline: 2.1774x; 1.0006x over previous
import functools

import jax
import jax.numpy as jnp
from jax import lax
from jax.experimental import pallas as pl
from jax.experimental.pallas import tpu as pltpu
from jax.experimental.pallas import tpu_sc as plsc

VOCAB = 100000
D_MODEL = 64

_NC = 2
_NS = 16
_NW = _NC * _NS

_B = 4096 * 200
_B_PER_W = _B // _NW
_CHUNK = 256
_N_CHUNK = _B_PER_W // _CHUNK
_NBUF = 5
_N_OUTER = _N_CHUNK // _NBUF


def _emb_body(idx_hbm, table_hbm, out_hbm, idx_v, rows_v, gsem, osem):
    wid = lax.axis_index("s") * _NC + lax.axis_index("c")
    base = wid * _B_PER_W

    def start_gather(c, b):
        off = base + c * _CHUNK
        pltpu.sync_copy(idx_hbm.at[pl.ds(off, _CHUNK)], idx_v.at[b])
        pltpu.async_copy(table_hbm.at[idx_v.at[b]], rows_v.at[b], gsem.at[b])

    def wait_gather(b):
        pltpu.make_async_copy(
            table_hbm.at[idx_v.at[b]], rows_v.at[b], gsem.at[b]).wait()

    def start_store(c, b):
        off = base + c * _CHUNK
        pltpu.async_copy(rows_v.at[b], out_hbm.at[pl.ds(off, _CHUNK)],
                         osem.at[b])

    def wait_store(c, b):
        off = base + c * _CHUNK
        pltpu.make_async_copy(
            rows_v.at[b], out_hbm.at[pl.ds(off, _CHUNK)], osem.at[b]).wait()

    for b in range(_NBUF):
        start_gather(b, b)

    def outer(o, carry):
        c0 = o * _NBUF
        for b in range(_NBUF):
            wait_gather(b)
            start_store(c0 + b, b)
        for b in range(_NBUF):
            wait_store(c0 + b, b)
            start_gather(c0 + b + _NBUF, b)
        return carry

    lax.fori_loop(0, _N_OUTER - 1, outer, 0)

    c0 = (_N_OUTER - 1) * _NBUF
    for b in range(_NBUF):
        wait_gather(b)
        start_store(c0 + b, b)
    for b in range(_NBUF):
        wait_store(c0 + b, b)


_emb = functools.partial(
    pl.kernel,
    out_type=jax.ShapeDtypeStruct((_B, D_MODEL), jnp.float32),
    mesh=plsc.VectorSubcoreMesh(core_axis_name="c", subcore_axis_name="s"),
    scratch_types=[
        pltpu.VMEM((_NBUF, _CHUNK), jnp.int32),
        pltpu.VMEM((_NBUF, _CHUNK, D_MODEL), jnp.float32),
        pltpu.SemaphoreType.DMA((_NBUF,)),
        pltpu.SemaphoreType.DMA((_NBUF,)),
    ],
    compiler_params=pltpu.CompilerParams(use_tc_tiling_on_sc=False),
)(_emb_body)


def kernel(sequence, lookup):
    idx = sequence.reshape(-1).astype(jnp.int32)
    out = _emb(idx, lookup)
    return out.reshape(sequence.shape + (D_MODEL,))
